# baseline (speedup 1.0000x reference)
import functools

import jax
import jax.numpy as jnp
from jax import lax
from jax.experimental import pallas as pl
from jax.experimental.pallas import tpu as pltpu
from jax.experimental.pallas import tpu_sc as plsc

N_CAT = 1000
BATCH = 16384
TOTAL = BATCH * N_CAT

ZROWS = 512
NZCOPIES = BATCH // ZROWS


def _zero_fill_body(o_hbm, zbuf, sem):
    zbuf[...] = jnp.zeros_like(zbuf)
    copies = []
    for i in range(NZCOPIES):
        copies.append(
            pltpu.async_copy(zbuf, o_hbm.at[pl.ds(i * ZROWS, ZROWS)], sem)
        )
    for c in copies:
        c.wait()


_zero_fill = pl.pallas_call(
    _zero_fill_body,
    out_shape=jax.ShapeDtypeStruct((BATCH, N_CAT), jnp.float32),
    out_specs=pl.BlockSpec(memory_space=pl.ANY),
    scratch_shapes=[
        pltpu.VMEM((ZROWS, N_CAT), jnp.float32),
        pltpu.SemaphoreType.DMA,
    ],
)

NUM_CORES = 2
NUM_SUBCORES = 16
LANES = 16
NUM_WORKERS = NUM_CORES * NUM_SUBCORES
ROWS_PER_W = BATCH // NUM_WORKERS
SCAT_W = 128
NSCAT = ROWS_PER_W // SCAT_W

_mesh = plsc.VectorSubcoreMesh(core_axis_name="c", subcore_axis_name="s")


@functools.partial(
    pl.kernel,
    mesh=_mesh,
    compiler_params=pltpu.CompilerParams(needs_layout_passes=False),
    scratch_types=[
        pltpu.VMEM((ROWS_PER_W,), jnp.int32),
        pltpu.VMEM((NSCAT, SCAT_W), jnp.int32),
        pltpu.VMEM((SCAT_W,), jnp.float32),
        pltpu.SemaphoreType.DMA,
    ],
)
def _scatter_ones(x_hbm, out_ref, xb, idxb, ones_v, sem):
    wid = lax.axis_index("s") * NUM_CORES + lax.axis_index("c")
    base_row = wid * ROWS_PER_W

    pltpu.sync_copy(x_hbm.at[pl.ds(base_row, ROWS_PER_W)], xb)

    ones = jnp.ones((LANES,), jnp.float32)
    lane = lax.iota(jnp.int32, LANES)

    for i in range(ROWS_PER_W // LANES):
        row = base_row + i * LANES + lane
        xv = xb[pl.ds(i * LANES, LANES)]
        idxb[i // (SCAT_W // LANES), pl.ds((i % (SCAT_W // LANES)) * LANES, LANES)] = (
            row * N_CAT + xv
        )
        if i < SCAT_W // LANES:
            ones_v[pl.ds(i * LANES, LANES)] = ones

    copies = []
    for g in range(NSCAT):
        copies.append(pltpu.async_copy(ones_v, out_ref.at[idxb.at[g]], sem))
    for c in copies:
        c.wait()


def kernel(x):
    zeros2d = _zero_fill()
    buf = jax.new_ref(zeros2d.reshape(TOTAL))
    _scatter_ones(x.astype(jnp.int32), buf)
    return buf[...].reshape(BATCH, N_CAT)


# speedup vs baseline: 1.4362x; 1.4362x over previous
import functools

import jax
import jax.numpy as jnp
from jax import lax
from jax.experimental import pallas as pl
from jax.experimental.pallas import tpu as pltpu
from jax.experimental.pallas import tpu_sc as plsc

N_CAT = 1000
BATCH = 16384
TOTAL = BATCH * N_CAT

NUM_CORES = 2
NUM_SUBCORES = 16
LANES = 16
NUM_WORKERS = NUM_CORES * NUM_SUBCORES
ROWS_PER_W = BATCH // NUM_WORKERS
WORDS_PER_W = ROWS_PER_W * N_CAT

ZWIN = 256000
NZDMA = WORDS_PER_W // ZWIN
ZSLICE = ZWIN // NUM_SUBCORES

SCAT_W = 128
NSCAT = ROWS_PER_W // SCAT_W

_mesh = plsc.VectorSubcoreMesh(core_axis_name="c", subcore_axis_name="s")


@functools.partial(
    pl.kernel,
    out_type=jax.ShapeDtypeStruct((TOTAL,), jnp.float32),
    mesh=_mesh,
    compiler_params=pltpu.CompilerParams(needs_layout_passes=False),
    scratch_types=[
        pltpu.VMEM_SHARED((ZWIN,), jnp.float32),
        pltpu.VMEM((ZSLICE,), jnp.float32),
        pltpu.VMEM((ROWS_PER_W,), jnp.int32),
        pltpu.VMEM((NSCAT, SCAT_W), jnp.int32),
        pltpu.VMEM((SCAT_W,), jnp.float32),
        pltpu.SemaphoreType.DMA,
        pltpu.SemaphoreType.DMA,
    ],
)
def _onehot_sc(x_hbm, out_hbm, zwin, zslice, xb, idxb, ones_v, zsem, ssem):
    cid = lax.axis_index("c")
    sid = lax.axis_index("s")
    wid = sid * NUM_CORES + cid
    base_row = wid * ROWS_PER_W
    base_word = wid * WORDS_PER_W

    zeros = jnp.zeros((LANES,), jnp.float32)
    ones = jnp.ones((LANES,), jnp.float32)
    lane = lax.iota(jnp.int32, LANES)

    def _zero(i, _):
        zslice[pl.ds(i * LANES, LANES)] = zeros
        return 0

    lax.fori_loop(0, ZSLICE // LANES, _zero, 0, unroll=8)
    pltpu.sync_copy(zslice, zwin.at[pl.ds(sid * ZSLICE, ZSLICE)])

    pltpu.sync_copy(x_hbm.at[pl.ds(base_row, ROWS_PER_W)], xb)

    plsc.subcore_barrier()

    zcopies = []
    for k in range(NZDMA):
        zcopies.append(
            pltpu.async_copy(
                zwin, out_hbm.at[pl.ds(base_word + k * ZWIN, ZWIN)], zsem
            )
        )

    for i in range(ROWS_PER_W // LANES):
        row = base_row + i * LANES + lane
        xv = xb[pl.ds(i * LANES, LANES)]
        idxb[i // (SCAT_W // LANES), pl.ds((i % (SCAT_W // LANES)) * LANES, LANES)] = (
            row * N_CAT + xv
        )
        if i < SCAT_W // LANES:
            ones_v[pl.ds(i * LANES, LANES)] = ones

    for c in zcopies:
        c.wait()

    scopies = []
    for g in range(NSCAT):
        scopies.append(pltpu.async_copy(ones_v, out_hbm.at[idxb.at[g]], ssem))
    for c in scopies:
        c.wait()


def kernel(x):
    return _onehot_sc(x.astype(jnp.int32)).reshape(BATCH, N_CAT)


# speedup vs baseline: 1.5674x; 1.0914x over previous
import functools

import jax
import jax.numpy as jnp
from jax import lax
from jax.experimental import pallas as pl
from jax.experimental.pallas import tpu as pltpu
from jax.experimental.pallas import tpu_sc as plsc

N_CAT = 1000
BATCH = 16384
NUM_CORES = 2
NUM_SUBCORES = 16
LANES = 16
NUM_WORKERS = NUM_CORES * NUM_SUBCORES
ROWS_PER_W = BATCH // NUM_WORKERS
CHUNK_ROWS = 32
CHUNK_WORDS = CHUNK_ROWS * N_CAT
NUM_CHUNKS = ROWS_PER_W // CHUNK_ROWS

_mesh = plsc.VectorSubcoreMesh(core_axis_name="c", subcore_axis_name="s")


@functools.partial(
    pl.kernel,
    out_type=jax.ShapeDtypeStruct((BATCH * N_CAT,), jnp.float32),
    mesh=_mesh,
    compiler_params=pltpu.CompilerParams(needs_layout_passes=False),
    scratch_types=[
        pltpu.VMEM((ROWS_PER_W,), jnp.int32),
        pltpu.VMEM((CHUNK_WORDS,), jnp.float32),
        pltpu.VMEM((CHUNK_WORDS,), jnp.float32),
        pltpu.SemaphoreType.DMA,
        pltpu.SemaphoreType.DMA,
    ],
)
def _onehot_sc(x_hbm, out_hbm, idx_v, buf_a, buf_b, sem_a, sem_b):
    wid = lax.axis_index("s") * NUM_CORES + lax.axis_index("c")
    base_row = wid * ROWS_PER_W

    pltpu.sync_copy(x_hbm.at[pl.ds(base_row * 1, ROWS_PER_W)], idx_v)

    zeros = jnp.zeros((LANES,), jnp.float32)
    ones = jnp.ones((LANES,), jnp.float32)
    lane_off = lax.iota(jnp.int32, LANES) * N_CAT

    def _zero(i, _):
        buf_a[pl.ds(i * LANES, LANES)] = zeros
        buf_b[pl.ds(i * LANES, LANES)] = zeros
        return 0

    lax.fori_loop(0, CHUNK_WORDS // LANES, _zero, 0, unroll=8)

    bufs = (buf_a, buf_b)
    sems = (sem_a, sem_b)

    def chunk_flat_indices(g):
        flats = []
        for t in range(CHUNK_ROWS // LANES):
            xv = idx_v[pl.ds(g * CHUNK_ROWS + t * LANES, LANES)]
            flats.append(lane_off + (t * LANES * N_CAT) + xv)
        return flats

    inflight = [None, None]
    for g in range(NUM_CHUNKS):
        b = g % 2
        buf = bufs[b]
        if inflight[b] is not None:
            copy, old_flats = inflight[b]
            copy.wait()
            for fv in old_flats:
                plsc.store_scatter(buf, [fv], zeros)
        flats = chunk_flat_indices(g)
        for fv in flats:
            plsc.store_scatter(buf, [fv], ones)
        dst = out_hbm.at[pl.ds((base_row + g * CHUNK_ROWS) * N_CAT, CHUNK_WORDS)]
        copy = pltpu.async_copy(buf, dst, sems[b])
        inflight[b] = (copy, flats)

    for b in range(2):
        if inflight[b] is not None:
            inflight[b][0].wait()


def kernel(x):
    out_flat = _onehot_sc(x.astype(jnp.int32))
    return out_flat.reshape(BATCH, N_CAT)
